# baseline (speedup 1.0000x reference)
import jax
import jax.numpy as jnp
from jax import lax
from jax.experimental import pallas as pl
from jax.experimental.pallas import tpu as pltpu
from jax.experimental.pallas import tpu_sc as plsc

B = 16384
DIM = 64
PDIM = 128
NUM_CORES = 2
NUM_SUBCORES = 16
NW = NUM_CORES * NUM_SUBCORES
BPW = B // NW
CHUNK = 128
NCHUNK = BPW // CHUNK
LANES = 16


def _gather_body(rt_ref, ln_ref, tp_ref, wr_ref, wl_ref, wt_ref, out_ref,
                 i0a, i1a, i2a, i0b, i1b, i2b,
                 r0a, r1a, r2a, r0b, r1b, r2b,
                 outT, sem_a, sem_b):
    wid = lax.axis_index("s") * NUM_CORES + lax.axis_index("c")
    base = wid * BPW
    irefs = (rt_ref, ln_ref, tp_ref)
    tabs = (wr_ref, wl_ref, wt_ref)
    idxs = ((i0a, i1a, i2a), (i0b, i1b, i2b))
    rows = ((r0a, r1a, r2a), (r0b, r1b, r2b))
    sems = (sem_a, sem_b)

    def fire(k):
        s = k % 2
        b = base + k * CHUNK
        for c in range(3):
            pltpu.sync_copy(irefs[c].at[pl.ds(b, CHUNK)], idxs[s][c])
        return [pltpu.async_copy(tabs[c].at[idxs[s][c]], rows[s][c], sems[s])
                for c in range(3)]

    def process(k, descs):
        s = k % 2
        for d in descs:
            d.wait()

        def f_body(f, carry):
            cidx = jnp.full((LANES,), f, jnp.int32)
            for c in range(3):
                for g in range(CHUNK // LANES):
                    ridx = lax.iota(jnp.int32, LANES) + (LANES * g)
                    v = plsc.load_gather(rows[s][c], [ridx, cidx])
                    outT[DIM * c + f, pl.ds(LANES * g, LANES)] = v
            return carry

        lax.fori_loop(0, DIM, f_body, 0)
        b = base + k * CHUNK
        pltpu.sync_copy(outT, out_ref.at[:, pl.ds(b, CHUNK)])

    descs = fire(0)
    for k in range(NCHUNK):
        nxt = fire(k + 1) if k + 1 < NCHUNK else None
        process(k, descs)
        descs = nxt


@jax.jit
def _lookup_concat(road_type, lane, time_period, W_road, W_lane, W_time):
    pad = [(0, 0), (0, PDIM - DIM)]
    wr = jnp.pad(W_road, pad)
    wl = jnp.pad(W_lane, pad)
    wt = jnp.pad(W_time, pad)

    mesh = plsc.VectorSubcoreMesh(core_axis_name="c", subcore_axis_name="s")
    out_t = pl.kernel(
        _gather_body,
        out_type=jax.ShapeDtypeStruct((3 * DIM, B), jnp.float32),
        mesh=mesh,
        compiler_params=pltpu.CompilerParams(needs_layout_passes=False),
        scratch_types=[
            pltpu.VMEM((CHUNK,), jnp.int32),
            pltpu.VMEM((CHUNK,), jnp.int32),
            pltpu.VMEM((CHUNK,), jnp.int32),
            pltpu.VMEM((CHUNK,), jnp.int32),
            pltpu.VMEM((CHUNK,), jnp.int32),
            pltpu.VMEM((CHUNK,), jnp.int32),
            pltpu.VMEM((CHUNK, PDIM), jnp.float32),
            pltpu.VMEM((CHUNK, PDIM), jnp.float32),
            pltpu.VMEM((CHUNK, PDIM), jnp.float32),
            pltpu.VMEM((CHUNK, PDIM), jnp.float32),
            pltpu.VMEM((CHUNK, PDIM), jnp.float32),
            pltpu.VMEM((CHUNK, PDIM), jnp.float32),
            pltpu.VMEM((3 * DIM, CHUNK), jnp.float32),
            pltpu.SemaphoreType.DMA,
            pltpu.SemaphoreType.DMA,
        ],
    )(road_type, lane, time_period, wr, wl, wt)
    return out_t.T


def kernel(road_type, lane, time_period, W_road, W_lane, W_time):
    return _lookup_concat(
        road_type.astype(jnp.int32),
        lane.astype(jnp.int32),
        time_period.astype(jnp.int32),
        W_road, W_lane, W_time,
    )


# speedup vs baseline: 1.7883x; 1.7883x over previous
import jax
import jax.numpy as jnp
from jax import lax
from jax.experimental import pallas as pl
from jax.experimental.pallas import tpu as pltpu
from jax.experimental.pallas import tpu_sc as plsc

B = 16384
DIM = 64
PDIM = 128
NUM_CORES = 2
NUM_SUBCORES = 16
NW = NUM_CORES * NUM_SUBCORES
BPW = B // NW
CHUNK = 128
NCHUNK = BPW // CHUNK


def _gather_body(rt_ref, ln_ref, tp_ref, w01_ref, w1_ref, w2_ref, out_ref,
                 i0a, i1a, i2a, i0b, i1b, i2b,
                 ab_a, c_a, ab_b, c_b,
                 sem0a, sem1a, sem2a, sem0b, sem1b, sem2b):
    wid = lax.axis_index("s") * NUM_CORES + lax.axis_index("c")
    base = wid * BPW
    idxs = ((i0a, i1a, i2a), (i0b, i1b, i2b))
    bufs = ((ab_a, c_a), (ab_b, c_b))
    sems = ((sem0a, sem1a, sem2a), (sem0b, sem1b, sem2b))

    def stage1(k):
        s = k % 2
        b = base + k * CHUNK
        pltpu.sync_copy(rt_ref.at[pl.ds(b, CHUNK)], idxs[s][0])
        pltpu.sync_copy(ln_ref.at[pl.ds(b, CHUNK)], idxs[s][1])
        pltpu.sync_copy(tp_ref.at[pl.ds(b, CHUNK)], idxs[s][2])
        g0 = pltpu.async_copy(w01_ref.at[idxs[s][0]], bufs[s][0], sems[s][0])
        g2 = pltpu.async_copy(w2_ref.at[idxs[s][2]], bufs[s][1], sems[s][2])
        return g0, g2

    def stage2(k, g0):
        s = k % 2
        g0.wait()
        return pltpu.async_copy(w1_ref.at[idxs[s][1]], bufs[s][0], sems[s][1],
                                add=True)

    def stage3(k, g1, g2):
        s = k % 2
        b = base + k * CHUNK
        g1.wait()
        pltpu.sync_copy(bufs[s][0], out_ref.at[pl.ds(b, CHUNK), pl.ds(0, PDIM)])
        g2.wait()
        pltpu.sync_copy(bufs[s][1],
                        out_ref.at[pl.ds(b, CHUNK), pl.ds(PDIM, PDIM)])

    g0, g2 = stage1(0)
    for k in range(NCHUNK):
        g1 = stage2(k, g0)
        if k + 1 < NCHUNK:
            g0_next, g2_next = stage1(k + 1)
        stage3(k, g1, g2)
        if k + 1 < NCHUNK:
            g0, g2 = g0_next, g2_next


@jax.jit
def _lookup_concat(road_type, lane, time_period, W_road, W_lane, W_time):
    w01 = jnp.pad(W_road, [(0, 0), (0, PDIM - DIM)])
    w1 = jnp.pad(W_lane, [(0, 0), (PDIM - DIM, 0)])
    w2 = jnp.pad(W_time, [(0, 0), (0, PDIM - DIM)])

    mesh = plsc.VectorSubcoreMesh(core_axis_name="c", subcore_axis_name="s")
    out_wide = pl.kernel(
        _gather_body,
        out_type=jax.ShapeDtypeStruct((B, 2 * PDIM), jnp.float32),
        mesh=mesh,
        scratch_types=[
            pltpu.VMEM((CHUNK,), jnp.int32),
            pltpu.VMEM((CHUNK,), jnp.int32),
            pltpu.VMEM((CHUNK,), jnp.int32),
            pltpu.VMEM((CHUNK,), jnp.int32),
            pltpu.VMEM((CHUNK,), jnp.int32),
            pltpu.VMEM((CHUNK,), jnp.int32),
            pltpu.VMEM((CHUNK, PDIM), jnp.float32),
            pltpu.VMEM((CHUNK, PDIM), jnp.float32),
            pltpu.VMEM((CHUNK, PDIM), jnp.float32),
            pltpu.VMEM((CHUNK, PDIM), jnp.float32),
            pltpu.SemaphoreType.DMA,
            pltpu.SemaphoreType.DMA,
            pltpu.SemaphoreType.DMA,
            pltpu.SemaphoreType.DMA,
            pltpu.SemaphoreType.DMA,
            pltpu.SemaphoreType.DMA,
        ],
    )(road_type, lane, time_period, w01, w1, w2)
    return out_wide[:, :3 * DIM]


def kernel(road_type, lane, time_period, W_road, W_lane, W_time):
    return _lookup_concat(
        road_type.astype(jnp.int32),
        lane.astype(jnp.int32),
        time_period.astype(jnp.int32),
        W_road, W_lane, W_time,
    )
